# baseline (speedup 1.0000x reference)
import functools
import math

import jax
import jax.numpy as jnp
from jax import lax
from jax.experimental import pallas as pl
from jax.experimental.pallas import tpu as pltpu
from jax.experimental.pallas import tpu_sc as plsc

_BN_EPS = 1e-5
_BN_S = 1.0 / math.sqrt(1.0 + _BN_EPS)

_NC = 2
_NS = 16
_NW = _NC * _NS

_CHUNK = 128
_ZROWS = 120


def _agg_body(n_nodes, n_acc, emb, n_chunks,
              x_hbm, src_hbm, dst_hbm, out_hbm,
              acc, dst_i, sp0, sp1, buf0, buf1, sem_i, isem0, isem1, sem0, sem1):
    cid = lax.axis_index("c")
    sid = lax.axis_index("s")
    wid = sid * _NC + cid

    e_per_tile = n_chunks * _CHUNK
    ebase = wid * e_per_tile

    cp_d = pltpu.async_copy(dst_hbm.at[wid], dst_i, sem_i)

    def _isl(j, sp, isem):
        pltpu.async_copy(src_hbm.at[pl.ds(ebase + j * _CHUNK, _CHUNK)], sp, isem)

    def _isw(sp, isem):
        pltpu.make_async_copy(src_hbm.at[pl.ds(0, _CHUNK)], sp, isem).wait()

    def _gather(sp, buf, sem):
        return pltpu.async_copy(x_hbm.at[sp], buf, sem)

    def _gwait(buf, sem):
        pltpu.make_async_copy(x_hbm.at[sp0], buf, sem).wait()

    def _scat(j, buf):
        pltpu.sync_copy(buf, acc.at[dst_i.at[j]], add=True)

    _isl(0, sp0, isem0)
    _isl(1, sp1, isem1)

    def _z(i, carry):
        r = i // (emb // 16)
        c = (i % (emb // 16)) * 16
        buf0[r, pl.ds(c, 16)] = jnp.zeros((16,), jnp.float32)
        return carry
    lax.fori_loop(0, _CHUNK * (emb // 16), _z, 0)

    rows_per_tile = n_acc // _NS
    nfull = rows_per_tile // _CHUNK
    rem = rows_per_tile - nfull * _CHUNK
    for k in range(nfull):
        pltpu.sync_copy(buf0, acc.at[pl.ds(sid * rows_per_tile + k * _CHUNK, _CHUNK)])
    if rem:
        pltpu.sync_copy(buf0.at[pl.ds(0, rem)],
                        acc.at[pl.ds(sid * rows_per_tile + nfull * _CHUNK, rem)])
    cp_d.wait()
    plsc.subcore_barrier()

    _isw(sp0, isem0)
    _gather(sp0, buf0, sem0)

    def _pair(k, carry):
        j = 2 * k
        _isw(sp1, isem1)
        _gather(sp1, buf1, sem1)
        _gwait(buf0, sem0)
        _isl(j + 2, sp0, isem0)
        _scat(j, buf0)
        _isw(sp0, isem0)
        _gather(sp0, buf0, sem0)
        _gwait(buf1, sem1)
        _scat(j + 1, buf1)
        _isl(j + 3, sp1, isem1)
        return carry
    lax.fori_loop(0, (n_chunks - 3) // 2, _pair, 0)
    _isw(sp1, isem1)
    _gather(sp1, buf1, sem1)
    _gwait(buf0, sem0)
    _isl(n_chunks - 1, sp0, isem0)
    _scat(n_chunks - 3, buf0)
    _isw(sp0, isem0)
    _gather(sp0, buf0, sem0)
    _gwait(buf1, sem1)
    _scat(n_chunks - 2, buf1)
    _gwait(buf0, sem0)
    _scat(n_chunks - 1, buf0)
    plsc.subcore_barrier()

    last_rows = n_nodes - (_NS - 1) * rows_per_tile
    @pl.when(sid < _NS - 1)
    def _():
        pltpu.sync_copy(acc.at[pl.ds(sid * rows_per_tile, rows_per_tile)],
                        out_hbm.at[cid, pl.ds(sid * rows_per_tile, rows_per_tile)])
    @pl.when(sid == _NS - 1)
    def _():
        pltpu.sync_copy(acc.at[pl.ds((_NS - 1) * rows_per_tile, last_rows)],
                        out_hbm.at[cid, pl.ds((_NS - 1) * rows_per_tile, last_rows)])


@functools.partial(jax.jit, static_argnames=("n_nodes", "emb", "e_pad"))
def _sc_agg(x, src, dst, *, n_nodes, emb, e_pad):
    n_acc = ((n_nodes + _NS * 8 - 1) // (_NS * 8)) * (_NS * 8)
    if n_acc == n_nodes:
        n_acc += _NS * 8
    e_per_tile = e_pad // _NW
    n_chunks = e_per_tile // _CHUNK
    body = functools.partial(_agg_body, n_nodes, n_acc, emb, n_chunks)
    dst3 = dst.reshape(_NW, n_chunks, _CHUNK)
    return pl.kernel(
        body,
        out_type=jax.ShapeDtypeStruct((_NC, n_nodes, emb), jnp.float32),
        mesh=plsc.VectorSubcoreMesh(core_axis_name="c", subcore_axis_name="s"),
        scratch_types=[
            pltpu.VMEM_SHARED((n_acc, emb), jnp.float32),
            pltpu.VMEM((n_chunks, _CHUNK), jnp.int32),
            pltpu.VMEM((_CHUNK,), jnp.int32),
            pltpu.VMEM((_CHUNK,), jnp.int32),
            pltpu.VMEM((_CHUNK, emb), jnp.float32),
            pltpu.VMEM((_CHUNK, emb), jnp.float32),
            pltpu.SemaphoreType.DMA,
            pltpu.SemaphoreType.DMA,
            pltpu.SemaphoreType.DMA,
            pltpu.SemaphoreType.DMA,
            pltpu.SemaphoreType.DMA,
        ],
    )(x, src, dst3)


_DN_T = (((1,), (1,)), ((), ()))


def _mlp_body(out_relu, eps_ref, x_ref, p_ref, w1_ref, b1_ref, w2_ref, b2_ref, o_ref):
    h = (1.0 + eps_ref[0]) * x_ref[...] + p_ref[0] + p_ref[1]
    t = lax.dot_general(h, w1_ref[...], _DN_T,
                        preferred_element_type=jnp.float32) + b1_ref[...]
    t = jnp.maximum(t * _BN_S, 0.0)
    o = lax.dot_general(t, w2_ref[...], _DN_T,
                        preferred_element_type=jnp.float32) + b2_ref[...]
    if out_relu:
        o = jnp.maximum(o * _BN_S, 0.0)
    o_ref[...] = o


@functools.partial(jax.jit, static_argnames=("out_relu", "blk"))
def _tc_mlp(x, p, w1, b1, w2, b2, eps, *, out_relu, blk):
    n, d = x.shape
    dh = w1.shape[0]
    grid = (n // blk,)
    return pl.pallas_call(
        functools.partial(_mlp_body, out_relu),
        grid=grid,
        in_specs=[
            pl.BlockSpec(memory_space=pltpu.SMEM),
            pl.BlockSpec((blk, d), lambda i: (i, 0)),
            pl.BlockSpec((_NC, blk, d), lambda i: (0, i, 0)),
            pl.BlockSpec((dh, d), lambda i: (0, 0)),
            pl.BlockSpec((dh,), lambda i: (0,)),
            pl.BlockSpec((d, dh), lambda i: (0, 0)),
            pl.BlockSpec((d,), lambda i: (0,)),
        ],
        out_specs=pl.BlockSpec((blk, d), lambda i: (i, 0)),
        out_shape=jax.ShapeDtypeStruct((n, d), jnp.float32),
    )(eps, x, p, w1, b1, w2, b2)


def kernel(x, edge_index, params):
    n, d = x.shape
    e = edge_index.shape[1]
    src = edge_index[0]
    dst = edge_index[1]
    step = _NW * _CHUNK
    nsteps = (e + step - 1) // step
    if nsteps < 5:
        nsteps = 5
    if nsteps % 2 == 0:
        nsteps += 1
    e_pad = nsteps * step
    if e_pad != e:
        n_acc = ((n + _NS * 8 - 1) // (_NS * 8)) * (_NS * 8)
        spare = (n_acc - n) if n_acc != n else _NS * 8
        npad = e_pad - e
        src = jnp.concatenate([src, jnp.arange(npad, dtype=jnp.int32) % n])
        dst = jnp.concatenate(
            [dst, n + (jnp.arange(npad, dtype=jnp.int32) % spare)])

    h = x
    nl = len(params)
    for i in range(nl):
        p = params[i]
        partials = _sc_agg(h, src, dst, n_nodes=n, emb=d, e_pad=e_pad)
        h = _tc_mlp(h, partials, p["W1"], p["b1"], p["W2"], p["b2"],
                    p["eps"].reshape(1), out_relu=(i < nl - 1), blk=2000)
    return h


# speedup vs baseline: 1.0806x; 1.0806x over previous
import functools
import math

import jax
import jax.numpy as jnp
from jax import lax
from jax.experimental import pallas as pl
from jax.experimental.pallas import tpu as pltpu
from jax.experimental.pallas import tpu_sc as plsc

_BN_EPS = 1e-5
_BN_S = 1.0 / math.sqrt(1.0 + _BN_EPS)

_NC = 2
_NS = 16
_NW = _NC * _NS

_CHUNK = 96
_ZROWS = 120


def _agg_body(n_nodes, n_acc, emb, n_chunks,
              x_hbm, src_hbm, dst_hbm, out_hbm,
              acc, src_i, dst_i, buf0, buf1, sem_i, sem0, sem1):
    cid = lax.axis_index("c")
    sid = lax.axis_index("s")
    wid = sid * _NC + cid

    e_per_tile = n_chunks * _CHUNK

    cp_s = pltpu.async_copy(src_hbm.at[pl.ds(wid * e_per_tile, e_per_tile)],
                            src_i, sem_i)
    cp_d = pltpu.async_copy(dst_hbm.at[wid], dst_i, sem_i)

    def _gather(j, buf, sem):
        off = pl.multiple_of(j * _CHUNK, 8)
        return pltpu.async_copy(x_hbm.at[src_i.at[pl.ds(off, _CHUNK)]], buf, sem)

    def _gwait(buf, sem):
        pltpu.make_async_copy(x_hbm.at[src_i.at[pl.ds(0, _CHUNK)]], buf, sem).wait()

    def _scat(j, buf):
        pltpu.sync_copy(buf, acc.at[dst_i.at[j]], add=True)

    def _z(i, carry):
        r = i // (emb // 16)
        c = (i % (emb // 16)) * 16
        buf0[r, pl.ds(c, 16)] = jnp.zeros((16,), jnp.float32)
        return carry
    lax.fori_loop(0, _CHUNK * (emb // 16), _z, 0)

    rows_per_tile = n_acc // _NS
    nfull = rows_per_tile // _CHUNK
    rem = rows_per_tile - nfull * _CHUNK
    for k in range(nfull):
        pltpu.sync_copy(buf0, acc.at[pl.ds(sid * rows_per_tile + k * _CHUNK, _CHUNK)])
    if rem:
        pltpu.sync_copy(buf0.at[pl.ds(0, rem)],
                        acc.at[pl.ds(sid * rows_per_tile + nfull * _CHUNK, rem)])
    cp_s.wait()
    cp_d.wait()
    plsc.subcore_barrier()

    _gather(0, buf0, sem0)

    def _pair(k, carry):
        j = 2 * k
        _gather(j + 1, buf1, sem1)
        _gwait(buf0, sem0)
        _scat(j, buf0)
        _gather(j + 2, buf0, sem0)
        _gwait(buf1, sem1)
        _scat(j + 1, buf1)
        return carry
    lax.fori_loop(0, (n_chunks - 1) // 2, _pair, 0)
    _gwait(buf0, sem0)
    _scat(n_chunks - 1, buf0)
    plsc.subcore_barrier()

    last_rows = n_nodes - (_NS - 1) * rows_per_tile
    @pl.when(sid < _NS - 1)
    def _():
        pltpu.sync_copy(acc.at[pl.ds(sid * rows_per_tile, rows_per_tile)],
                        out_hbm.at[cid, pl.ds(sid * rows_per_tile, rows_per_tile)])
    @pl.when(sid == _NS - 1)
    def _():
        pltpu.sync_copy(acc.at[pl.ds((_NS - 1) * rows_per_tile, last_rows)],
                        out_hbm.at[cid, pl.ds((_NS - 1) * rows_per_tile, last_rows)])


@functools.partial(jax.jit, static_argnames=("n_nodes", "emb", "e_pad"))
def _sc_agg(x, src, dst, *, n_nodes, emb, e_pad):
    n_acc = ((n_nodes + _NS * 8 - 1) // (_NS * 8)) * (_NS * 8)
    if n_acc == n_nodes:
        n_acc += _NS * 8
    e_per_tile = e_pad // _NW
    n_chunks = e_per_tile // _CHUNK
    body = functools.partial(_agg_body, n_nodes, n_acc, emb, n_chunks)
    dst3 = dst.reshape(_NW, n_chunks, _CHUNK)
    return pl.kernel(
        body,
        out_type=jax.ShapeDtypeStruct((_NC, n_nodes, emb), jnp.float32),
        mesh=plsc.VectorSubcoreMesh(core_axis_name="c", subcore_axis_name="s"),
        scratch_types=[
            pltpu.VMEM_SHARED((n_acc, emb), jnp.float32),
            pltpu.VMEM((n_chunks * _CHUNK,), jnp.int32),
            pltpu.VMEM((n_chunks, _CHUNK), jnp.int32),
            pltpu.VMEM((_CHUNK, emb), jnp.float32),
            pltpu.VMEM((_CHUNK, emb), jnp.float32),
            pltpu.SemaphoreType.DMA,
            pltpu.SemaphoreType.DMA,
            pltpu.SemaphoreType.DMA,
        ],
    )(x, src, dst3)


_DN_T = (((1,), (1,)), ((), ()))


def _mlp_body(out_relu, eps_ref, x_ref, p_ref, w1_ref, b1_ref, w2_ref, b2_ref, o_ref):
    h = (1.0 + eps_ref[0]) * x_ref[...] + p_ref[0] + p_ref[1]
    t = lax.dot_general(h, w1_ref[...], _DN_T,
                        preferred_element_type=jnp.float32) + b1_ref[...]
    t = jnp.maximum(t * _BN_S, 0.0)
    o = lax.dot_general(t, w2_ref[...], _DN_T,
                        preferred_element_type=jnp.float32) + b2_ref[...]
    if out_relu:
        o = jnp.maximum(o * _BN_S, 0.0)
    o_ref[...] = o


@functools.partial(jax.jit, static_argnames=("out_relu", "blk"))
def _tc_mlp(x, p, w1, b1, w2, b2, eps, *, out_relu, blk):
    n, d = x.shape
    dh = w1.shape[0]
    grid = (n // blk,)
    return pl.pallas_call(
        functools.partial(_mlp_body, out_relu),
        grid=grid,
        in_specs=[
            pl.BlockSpec(memory_space=pltpu.SMEM),
            pl.BlockSpec((blk, d), lambda i: (i, 0)),
            pl.BlockSpec((_NC, blk, d), lambda i: (0, i, 0)),
            pl.BlockSpec((dh, d), lambda i: (0, 0)),
            pl.BlockSpec((dh,), lambda i: (0,)),
            pl.BlockSpec((d, dh), lambda i: (0, 0)),
            pl.BlockSpec((d,), lambda i: (0,)),
        ],
        out_specs=pl.BlockSpec((blk, d), lambda i: (i, 0)),
        out_shape=jax.ShapeDtypeStruct((n, d), jnp.float32),
    )(eps, x, p, w1, b1, w2, b2)


def kernel(x, edge_index, params):
    n, d = x.shape
    e = edge_index.shape[1]
    src = edge_index[0]
    dst = edge_index[1]
    step = _NW * _CHUNK
    nsteps = (e + step - 1) // step
    if nsteps < 5:
        nsteps = 5
    if nsteps % 2 == 0:
        nsteps += 1
    e_pad = nsteps * step
    if e_pad != e:
        n_acc = ((n + _NS * 8 - 1) // (_NS * 8)) * (_NS * 8)
        spare = (n_acc - n) if n_acc != n else _NS * 8
        npad = e_pad - e
        src = jnp.concatenate([src, jnp.arange(npad, dtype=jnp.int32) % n])
        dst = jnp.concatenate(
            [dst, n + (jnp.arange(npad, dtype=jnp.int32) % spare)])

    h = x
    nl = len(params)
    for i in range(nl):
        p = params[i]
        partials = _sc_agg(h, src, dst, n_nodes=n, emb=d, e_pad=e_pad)
        h = _tc_mlp(h, partials, p["W1"], p["b1"], p["W2"], p["b2"],
                    p["eps"].reshape(1), out_relu=(i < nl - 1), blk=2000)
    return h


# speedup vs baseline: 1.0981x; 1.0161x over previous
import functools
import math

import jax
import jax.numpy as jnp
from jax import lax
from jax.experimental import pallas as pl
from jax.experimental.pallas import tpu as pltpu
from jax.experimental.pallas import tpu_sc as plsc

_BN_EPS = 1e-5
_BN_S = 1.0 / math.sqrt(1.0 + _BN_EPS)

_NC = 2
_NS = 16
_NW = _NC * _NS

_CHUNK = 104
_ZROWS = 120


def _agg_body(n_nodes, n_acc, emb, n_chunks,
              x_hbm, src_hbm, dst_hbm, out_hbm,
              acc, src_i, dst_i, buf0, buf1, sem_i, sem0, sem1):
    cid = lax.axis_index("c")
    sid = lax.axis_index("s")
    wid = sid * _NC + cid

    e_per_tile = n_chunks * _CHUNK

    cp_s = pltpu.async_copy(src_hbm.at[pl.ds(wid * e_per_tile, e_per_tile)],
                            src_i, sem_i)
    cp_d = pltpu.async_copy(dst_hbm.at[wid], dst_i, sem_i)

    def _gather(j, buf, sem):
        off = pl.multiple_of(j * _CHUNK, 8)
        return pltpu.async_copy(x_hbm.at[src_i.at[pl.ds(off, _CHUNK)]], buf, sem)

    def _gwait(buf, sem):
        pltpu.make_async_copy(x_hbm.at[src_i.at[pl.ds(0, _CHUNK)]], buf, sem).wait()

    def _scat(j, buf):
        pltpu.sync_copy(buf, acc.at[dst_i.at[j]], add=True)

    def _z(i, carry):
        r = i // (emb // 16)
        c = (i % (emb // 16)) * 16
        buf0[r, pl.ds(c, 16)] = jnp.zeros((16,), jnp.float32)
        return carry
    lax.fori_loop(0, _CHUNK * (emb // 16), _z, 0)

    rows_per_tile = n_acc // _NS
    nfull = rows_per_tile // _CHUNK
    rem = rows_per_tile - nfull * _CHUNK
    for k in range(nfull):
        pltpu.sync_copy(buf0, acc.at[pl.ds(sid * rows_per_tile + k * _CHUNK, _CHUNK)])
    if rem:
        pltpu.sync_copy(buf0.at[pl.ds(0, rem)],
                        acc.at[pl.ds(sid * rows_per_tile + nfull * _CHUNK, rem)])
    cp_s.wait()
    cp_d.wait()
    plsc.subcore_barrier()

    _gather(0, buf0, sem0)

    def _pair(k, carry):
        j = 2 * k
        _gather(j + 1, buf1, sem1)
        _gwait(buf0, sem0)
        _scat(j, buf0)
        _gather(j + 2, buf0, sem0)
        _gwait(buf1, sem1)
        _scat(j + 1, buf1)
        return carry
    lax.fori_loop(0, (n_chunks - 1) // 2, _pair, 0)
    _gwait(buf0, sem0)
    _scat(n_chunks - 1, buf0)
    plsc.subcore_barrier()

    last_rows = n_nodes - (_NS - 1) * rows_per_tile
    @pl.when(sid < _NS - 1)
    def _():
        pltpu.sync_copy(acc.at[pl.ds(sid * rows_per_tile, rows_per_tile)],
                        out_hbm.at[cid, pl.ds(sid * rows_per_tile, rows_per_tile)])
    @pl.when(sid == _NS - 1)
    def _():
        pltpu.sync_copy(acc.at[pl.ds((_NS - 1) * rows_per_tile, last_rows)],
                        out_hbm.at[cid, pl.ds((_NS - 1) * rows_per_tile, last_rows)])


@functools.partial(jax.jit, static_argnames=("n_nodes", "emb", "e_pad"))
def _sc_agg(x, src, dst, *, n_nodes, emb, e_pad):
    n_acc = ((n_nodes + _NS * 8 - 1) // (_NS * 8)) * (_NS * 8)
    if n_acc == n_nodes:
        n_acc += _NS * 8
    e_per_tile = e_pad // _NW
    n_chunks = e_per_tile // _CHUNK
    body = functools.partial(_agg_body, n_nodes, n_acc, emb, n_chunks)
    dst3 = dst.reshape(_NW, n_chunks, _CHUNK)
    return pl.kernel(
        body,
        out_type=jax.ShapeDtypeStruct((_NC, n_nodes, emb), jnp.float32),
        mesh=plsc.VectorSubcoreMesh(core_axis_name="c", subcore_axis_name="s"),
        scratch_types=[
            pltpu.VMEM_SHARED((n_acc, emb), jnp.float32),
            pltpu.VMEM((n_chunks * _CHUNK,), jnp.int32),
            pltpu.VMEM((n_chunks, _CHUNK), jnp.int32),
            pltpu.VMEM((_CHUNK, emb), jnp.float32),
            pltpu.VMEM((_CHUNK, emb), jnp.float32),
            pltpu.SemaphoreType.DMA,
            pltpu.SemaphoreType.DMA,
            pltpu.SemaphoreType.DMA,
        ],
    )(x, src, dst3)


_DN_T = (((1,), (1,)), ((), ()))


def _mlp_body(out_relu, eps_ref, x_ref, p_ref, w1_ref, b1_ref, w2_ref, b2_ref, o_ref):
    h = (1.0 + eps_ref[0]) * x_ref[...] + p_ref[0] + p_ref[1]
    t = lax.dot_general(h, w1_ref[...], _DN_T,
                        preferred_element_type=jnp.float32) + b1_ref[...]
    t = jnp.maximum(t * _BN_S, 0.0)
    o = lax.dot_general(t, w2_ref[...], _DN_T,
                        preferred_element_type=jnp.float32) + b2_ref[...]
    if out_relu:
        o = jnp.maximum(o * _BN_S, 0.0)
    o_ref[...] = o


@functools.partial(jax.jit, static_argnames=("out_relu", "blk"))
def _tc_mlp(x, p, w1, b1, w2, b2, eps, *, out_relu, blk):
    n, d = x.shape
    dh = w1.shape[0]
    grid = (n // blk,)
    return pl.pallas_call(
        functools.partial(_mlp_body, out_relu),
        grid=grid,
        in_specs=[
            pl.BlockSpec(memory_space=pltpu.SMEM),
            pl.BlockSpec((blk, d), lambda i: (i, 0)),
            pl.BlockSpec((_NC, blk, d), lambda i: (0, i, 0)),
            pl.BlockSpec((dh, d), lambda i: (0, 0)),
            pl.BlockSpec((dh,), lambda i: (0,)),
            pl.BlockSpec((d, dh), lambda i: (0, 0)),
            pl.BlockSpec((d,), lambda i: (0,)),
        ],
        out_specs=pl.BlockSpec((blk, d), lambda i: (i, 0)),
        out_shape=jax.ShapeDtypeStruct((n, d), jnp.float32),
    )(eps, x, p, w1, b1, w2, b2)


def kernel(x, edge_index, params):
    n, d = x.shape
    e = edge_index.shape[1]
    src = edge_index[0]
    dst = edge_index[1]
    step = _NW * _CHUNK
    nsteps = (e + step - 1) // step
    if nsteps < 5:
        nsteps = 5
    if nsteps % 2 == 0:
        nsteps += 1
    e_pad = nsteps * step
    if e_pad != e:
        n_acc = ((n + _NS * 8 - 1) // (_NS * 8)) * (_NS * 8)
        spare = (n_acc - n) if n_acc != n else _NS * 8
        npad = e_pad - e
        src = jnp.concatenate([src, jnp.arange(npad, dtype=jnp.int32) % n])
        dst = jnp.concatenate(
            [dst, n + (jnp.arange(npad, dtype=jnp.int32) % spare)])

    h = x
    nl = len(params)
    for i in range(nl):
        p = params[i]
        partials = _sc_agg(h, src, dst, n_nodes=n, emb=d, e_pad=e_pad)
        h = _tc_mlp(h, partials, p["W1"], p["b1"], p["W2"], p["b2"],
                    p["eps"].reshape(1), out_relu=(i < nl - 1), blk=2000)
    return h
